# Initial kernel scaffold; baseline (speedup 1.0000x reference)
import jax
import jax.numpy as jnp
from jax.experimental import pallas as pl


def kernel(node_features, layer_positions, pe):
    raise NotImplementedError("write your pallas kernel here")


# speedup vs baseline: 2.0525x; 2.0525x over previous
import functools

import jax
import jax.numpy as jnp
from jax import lax
from jax.experimental import pallas as pl
from jax.experimental.pallas import tpu as pltpu
from jax.experimental.pallas import tpu_sc as plsc

HIDDEN = 128
CHUNK = 128
LANES = 16


def _sc_kernel_body(nf_hbm, pos_hbm, pe_hbm, out_hbm,
                    pos_v, idx_v, rows_v, nf_v, sem_g, sem_n):
    num_cores = 2
    wid = lax.axis_index("s") * num_cores + lax.axis_index("c")
    n_rows = nf_hbm.shape[0]
    n_chunks = n_rows // CHUNK
    n_workers = 32

    def do_chunk(c):
        base = c * CHUNK
        pltpu.sync_copy(pos_hbm.at[pl.ds(base, CHUNK)], pos_v)
        for i in range(CHUNK // LANES):
            p = pos_v[pl.ds(i * LANES, LANES)]
            idx = jnp.clip((p * 999.0).astype(jnp.int32), 0, 999)
            idx_v[pl.ds(i * LANES, LANES)] = idx
        g = pltpu.async_copy(pe_hbm.at[idx_v], rows_v, sem_g)
        n = pltpu.async_copy(nf_hbm.at[pl.ds(base, CHUNK)], nf_v, sem_n)
        g.wait()
        n.wait()

        def row_body(r, carry):
            for j in range(HIDDEN // LANES):
                plsc.addupdate(rows_v.at[r, pl.ds(j * LANES, LANES)],
                               nf_v[r, pl.ds(j * LANES, LANES)])
            return carry

        lax.fori_loop(0, CHUNK, row_body, 0)
        pltpu.sync_copy(rows_v, out_hbm.at[pl.ds(base, CHUNK)])

    def outer(k, carry):
        do_chunk(k * n_workers + wid)
        return carry

    lax.fori_loop(0, n_chunks // n_workers, outer, 0)
    rem = n_chunks % n_workers
    if rem:
        @pl.when(wid < rem)
        def _():
            do_chunk((n_chunks // n_workers) * n_workers + wid)


def _build_sc_call(n_rows):
    mesh = plsc.VectorSubcoreMesh(core_axis_name="c", subcore_axis_name="s")
    return pl.kernel(
        _sc_kernel_body,
        mesh=mesh,
        out_type=jax.ShapeDtypeStruct((n_rows, HIDDEN), jnp.float32),
        scratch_types=[
            pltpu.VMEM((CHUNK,), jnp.float32),
            pltpu.VMEM((CHUNK,), jnp.int32),
            pltpu.VMEM((CHUNK, HIDDEN), jnp.float32),
            pltpu.VMEM((CHUNK, HIDDEN), jnp.float32),
            pltpu.SemaphoreType.DMA,
            pltpu.SemaphoreType.DMA,
        ],
    )


def kernel(node_features, layer_positions, pe):
    b, n, h = node_features.shape
    nf = node_features.reshape(b * n, h)
    pos = layer_positions.reshape(b * n)
    table = pe[0]
    out = _build_sc_call(b * n)(nf, pos, table)
    return out.reshape(b, n, h)
